# baseline (speedup 1.0000x reference)
import functools

import jax
import jax.numpy as jnp
from jax import lax
from jax.experimental import pallas as pl
from jax.experimental.pallas import tpu as pltpu
from jax.experimental.pallas import tpu_sc as plsc

_N = 10000
_E = 320000
_D = 128
_HID = 256

_NC = 2
_NS = 16
_NW = _NC * _NS
_C = 80
_EP = 320000
_EW = _EP // _NW
_NCHUNK = _EW // _C
_GC = 5
_NGRP = _NCHUNK // _GC
_NBUF = 3
_NP = 10112
_RPT = _NP // _NS
_DEGW = 128

_BLK = 1000
_GRID = _N // _BLK


@functools.cache
def _sc_agg(width):
    mesh = plsc.VectorSubcoreMesh(core_axis_name="c", subcore_axis_name="s")

    def body(h_hbm, src_hbm, dst_hbm, zero_hbm, out_hbm,
             srcb, dstb, *bufs_and_sems):
        rows = bufs_and_sems[:_NBUF]
        acc_sh = bufs_and_sems[_NBUF]
        gs = bufs_and_sems[_NBUF + 1:2 * _NBUF + 1]
        ss = bufs_and_sems[2 * _NBUF + 1:3 * _NBUF + 1]
        isem = bufs_and_sems[3 * _NBUF + 1]
        c = lax.axis_index("c")
        s = lax.axis_index("s")
        wid = s * _NC + c
        sw = src_hbm.at[wid]
        dw = dst_hbm.at[wid]
        pltpu.sync_copy(sw.at[0], srcb.at[0])
        pltpu.sync_copy(dw.at[0], dstb.at[0])
        pltpu.sync_copy(zero_hbm.at[pl.ds(s * _RPT, _RPT)],
                        acc_sh.at[pl.ds(s * _RPT, _RPT)])
        plsc.subcore_barrier()

        def grp(g, carry):
            b = g % 2
            nb = (g + 1) % 2

            @pl.when(g < _NGRP - 1)
            def _prefetch():
                pltpu.async_copy(sw.at[g + 1], srcb.at[nb], isem)
                pltpu.async_copy(dw.at[g + 1], dstb.at[nb], isem)

            S = srcb.at[b]
            Dx = dstb.at[b]
            for j in range(min(_NBUF, _GC)):
                pltpu.async_copy(h_hbm.at[S.at[j]], rows[j % _NBUF],
                                 gs[j % _NBUF])
            for j in range(_GC):
                p = j % _NBUF
                pltpu.make_async_copy(h_hbm.at[S.at[j]], rows[p],
                                      gs[p]).wait()
                pltpu.async_copy(rows[p], acc_sh.at[Dx.at[j]], ss[p],
                                 add=True)
                if j + _NBUF < _GC:
                    pltpu.make_async_copy(rows[p], acc_sh.at[Dx.at[j]],
                                          ss[p]).wait()
                    pltpu.async_copy(h_hbm.at[S.at[j + _NBUF]], rows[p],
                                     gs[p])
            for j in range(max(0, _GC - _NBUF), _GC):
                pltpu.make_async_copy(rows[j % _NBUF], acc_sh.at[Dx.at[j]],
                                      ss[j % _NBUF]).wait()

            @pl.when(g < _NGRP - 1)
            def _prefetch_wait():
                pltpu.make_async_copy(sw.at[g + 1], srcb.at[nb],
                                      isem).wait()
                pltpu.make_async_copy(dw.at[g + 1], dstb.at[nb],
                                      isem).wait()

            return carry

        lax.fori_loop(0, _NGRP, grp, 0)
        plsc.subcore_barrier()
        pltpu.sync_copy(acc_sh.at[pl.ds(s * _RPT, _RPT)],
                        out_hbm.at[c].at[pl.ds(s * _RPT, _RPT)])

    return pl.kernel(
        body,
        out_type=jax.ShapeDtypeStruct((_NC, _NP, width), jnp.float32),
        mesh=mesh,
        scratch_types=(
            [pltpu.VMEM((2, _GC, _C), jnp.int32),
             pltpu.VMEM((2, _GC, _C), jnp.int32)]
            + [pltpu.VMEM((_C, width), jnp.float32)] * _NBUF
            + [pltpu.VMEM_SHARED((_NP, width), jnp.float32)]
            + [pltpu.SemaphoreType.DMA] * (2 * _NBUF + 1)
        ),
    )


_DROW = 80


@functools.cache
def _sc_deg():
    mesh = plsc.VectorSubcoreMesh(core_axis_name="c", subcore_axis_name="s")

    nwords = _DROW * _D

    def body(dst_hbm, out_hbm, dst_v, cnt_v):
        c = lax.axis_index("c")
        s = lax.axis_index("s")
        wid = s * _NC + c
        pltpu.sync_copy(dst_hbm.at[wid], dst_v)
        zero16 = jnp.zeros((16,), jnp.float32)

        def zero(i, carry):
            cnt_v[pl.ds(16 * i, 16)] = zero16
            return carry

        lax.fori_loop(0, nwords // 16, zero, 0)
        ones16 = jnp.full((16,), 1.0, jnp.float32)

        def chunk(k, carry):
            for m in range(_C // 16):
                idx = dst_v[k, pl.ds(16 * m, 16)]
                plsc.addupdate_scatter(cnt_v, [idx], ones16)
            return carry

        lax.fori_loop(0, _NCHUNK, chunk, 0)
        pltpu.sync_copy(cnt_v, out_hbm.at[pl.ds(wid * nwords, nwords)])

    return pl.kernel(
        body,
        out_type=jax.ShapeDtypeStruct((_NW * nwords,), jnp.float32),
        mesh=mesh,
        compiler_params=pltpu.CompilerParams(needs_layout_passes=False),
        scratch_types=[
            pltpu.VMEM((_NCHUNK, _C), jnp.int32),
            pltpu.VMEM((nwords,), jnp.float32),
        ],
    )


def _tc_degsum_body(dp, out):
    out[...] = jnp.sum(dp[...], axis=0)


def _tc_degsum(dp):
    return pl.pallas_call(
        _tc_degsum_body,
        grid=(1,),
        in_specs=[pl.BlockSpec((_NW, _DROW, _D), lambda i: (0, 0, 0))],
        out_specs=pl.BlockSpec((_DROW, _D), lambda i: (0, 0)),
        out_shape=jax.ShapeDtypeStruct((_DROW, _D), jnp.float32),
    )(dp.reshape(_NW, _DROW, _D))


def _lane0(shape):
    return lax.broadcasted_iota(jnp.int32, shape, 1) == 0


def _col0(a):
    return jnp.sum(jnp.where(_lane0(a.shape), a, 0.0), axis=-1, keepdims=True)


def _tailnorm(a):
    sq = jnp.where(_lane0(a.shape), 0.0, a * a)
    return jnp.sqrt(jnp.sum(sq, axis=-1, keepdims=True))


def _dot(a, b):
    return jnp.dot(a, b, preferred_element_type=jnp.float32)


def _l_expmap0(u):
    un = jnp.clip(_tailnorm(u), 1e-7, 5.0)
    e = jnp.exp(un)
    ei = 1.0 / e
    ch = 0.5 * (e + ei)
    sh = 0.5 * (e - ei)
    return jnp.where(_lane0(u.shape), ch, u * (sh / un))


def _l_logmap0(xx):
    x0 = jnp.maximum(_col0(xx), 1.0 + 1e-7)
    d = jnp.log(x0 + jnp.sqrt(x0 * x0 - 1.0))
    nr = jnp.maximum(_tailnorm(xx), 1e-7)
    return jnp.where(_lane0(xx.shape), 0.0, xx * (d / nr))


def _s_expmap0(u):
    import numpy as _np
    un = jnp.clip(_tailnorm(u), 1e-7, _np.pi - 1e-2)
    return jnp.where(_lane0(u.shape), jnp.cos(un), u * (jnp.sin(un) / un))


def _arccos(x):
    import numpy as _np
    ax = jnp.abs(x)
    small = ax <= 0.5
    z = jnp.where(small, x * x, 0.5 * (1.0 - ax))
    s = jnp.where(small, ax, jnp.sqrt(z))
    r = (z * (0.16666586697 + z * (-0.042743422091 + z * (-0.0086563630030)))
         / (1.0 + z * (-0.70662963390)))
    asn = s + s * r
    sgn_asn = jnp.where(x < 0.0, -asn, asn)
    small_th = jnp.float32(_np.pi / 2) - sgn_asn
    big_th = jnp.where(x < 0.0, jnp.float32(_np.pi) - 2.0 * asn, 2.0 * asn)
    return jnp.where(small, small_th, big_th)


def _s_logmap0(xx):
    x0 = jnp.clip(_col0(xx), -1.0 + 1e-6, 1.0 - 1e-6)
    th = _arccos(x0)
    nr = jnp.maximum(_tailnorm(xx), 1e-7)
    return jnp.where(_lane0(xx.shape), 0.0, xx * (th / nr))


def _l_proju(x, v):
    inner = jnp.sum(jnp.where(_lane0(x.shape), -x * v, x * v),
                    axis=-1, keepdims=True)
    return v + inner * x


def _l_transp0back(x, v):
    x0 = _col0(x)
    v0 = _col0(v)
    xo = x + jnp.where(_lane0(x.shape), 1.0, 0.0)
    return v + (v0 / (1.0 + x0)) * xo


def _s_proju(x, v):
    return v - jnp.sum(x * v, axis=-1, keepdims=True) * x


def _s_transp0back(x, v):
    x0 = _col0(x)
    v0 = _col0(v)
    xo = x + jnp.where(_lane0(x.shape), 1.0, 0.0)
    return v - (v0 / jnp.maximum(1.0 + x0, 1e-5)) * xo


def _mlp(t, W1, b1, W2, b2):
    return _dot(jnp.maximum(_dot(t, W1) + b1, 0.0), W2) + b2


def _node_spec():
    return pl.BlockSpec((_BLK, _D), lambda i: (i, 0))


def _pair_spec(width):
    return pl.BlockSpec((_NC, _BLK, width), lambda i: (0, i, 0))


def _full_spec(shape):
    nd = len(shape)
    return pl.BlockSpec(shape, lambda i: (0,) * nd)


def _tc1_body(tok, eW1, eb1, eW2, eb2, hW1, hb1, hW2, hb2,
              sW1, sb1, sW2, sb2, hE_o, hH_o, hS_o):
    t = tok[...]
    hE_o[...] = _mlp(t, eW1[...], eb1[...], eW2[...], eb2[...])
    hH_o[...] = _mlp(t, hW1[...], hb1[...], hW2[...], hb2[...])
    hS_o[...] = _mlp(t, sW1[...], sb1[...], sW2[...], sb2[...])


def _tc1(tok, eW1, eb1, eW2, eb2, hW1, hb1, hW2, hb2, sW1, sb1, sW2, sb2):
    ws = [eW1, eb1, eW2, eb2, hW1, hb1, hW2, hb2, sW1, sb1, sW2, sb2]
    return pl.pallas_call(
        _tc1_body,
        grid=(_GRID,),
        in_specs=[_node_spec()] + [_full_spec(w.shape) for w in ws],
        out_specs=[_node_spec()] * 3,
        out_shape=[jax.ShapeDtypeStruct((_N, _D), jnp.float32)] * 3,
    )(tok, *ws)


def _tc2_body(hE, hH, hS, aE, aH, aS, degn, Wh0,
              xE_o, xH_o, xS_o, u_o, dinv_o):
    dinv = 1.0 / (1.0 + degn[...])
    dinv_o[...] = jnp.broadcast_to(dinv, (_BLK, _D))
    xE_o[...] = (hE[...] + aE[...][0] + aE[...][1]) * dinv
    xH = _l_expmap0((hH[...] + aH[...][0] + aH[...][1]) * dinv)
    xH_o[...] = xH
    xS_o[...] = _s_expmap0((hS[...] + aS[...][0] + aS[...][1]) * dinv)
    u_o[...] = _dot(_l_logmap0(xH), Wh0[...])


def _tc2(hE, hH, hS, aE, aH, aS, degn, Wh0):
    return pl.pallas_call(
        _tc2_body,
        grid=(_GRID,),
        in_specs=[_node_spec()] * 3 + [_pair_spec(_D)] * 3
        + [pl.BlockSpec((_BLK, 1), lambda i: (i, 0)), _full_spec(Wh0.shape)],
        out_specs=[_node_spec()] * 5,
        out_shape=[jax.ShapeDtypeStruct((_N, _D), jnp.float32)] * 5,
    )(hE, hH, hS, aE, aH, aS, degn, Wh0)


def _tc3_body(xS, u, aU, dinv, WsA, WsB, xH_o, v_o):
    xH = _l_expmap0((u[...] + aU[...][0] + aU[...][1]) * dinv[...])
    xH_o[...] = xH
    v_o[...] = (_dot(_s_logmap0(xS[...]), WsA[...])
                + _dot(_l_logmap0(xH), WsB[...]))


def _tc3(xS, u, aU, dinv, WsA, WsB):
    return pl.pallas_call(
        _tc3_body,
        grid=(_GRID,),
        in_specs=[_node_spec(), _node_spec(), _pair_spec(_D), _node_spec(),
                  _full_spec(WsA.shape), _full_spec(WsB.shape)],
        out_specs=[_node_spec()] * 2,
        out_shape=[jax.ShapeDtypeStruct((_N, _D), jnp.float32)] * 2,
    )(xS, u, aU, dinv, WsA, WsB)


def _tc4_body(xE, xH, v, aV, dinv, P1a, P1b, P1c, pb1, P2, pb2, WhN,
              xE_o, xS_o, u_o):
    xS = _s_expmap0((v[...] + aV[...][0] + aV[...][1]) * dinv[...])
    xS_o[...] = xS
    xEi = xE[...]
    xHi = xH[...]
    HE = _l_transp0back(xHi, _l_proju(xHi, xEi))
    SE = _s_transp0back(xS, _s_proju(xS, xEi))
    hmid = jnp.maximum(_dot(xEi, P1a[...]) + _dot(HE, P1b[...])
                       + _dot(SE, P1c[...]) + pb1[...], 0.0)
    xE_o[...] = _dot(hmid, P2[...]) + pb2[...]
    u_o[...] = _dot(_l_logmap0(xHi), WhN[...])


def _tc4(xE, xH, v, aV, dinv, P1a, P1b, P1c, pb1, P2, pb2, WhN):
    ws = [P1a, P1b, P1c, pb1, P2, pb2, WhN]
    return pl.pallas_call(
        _tc4_body,
        grid=(_GRID,),
        in_specs=[_node_spec(), _node_spec(), _node_spec(), _pair_spec(_D),
                  _node_spec()] + [_full_spec(w.shape) for w in ws],
        out_specs=[_node_spec()] * 3,
        out_shape=[jax.ShapeDtypeStruct((_N, _D), jnp.float32)] * 3,
    )(xE, xH, v, aV, dinv, *ws)


def kernel(x, tokens, eW1, eb1, eW2, eb2, hW1, hb1, hW2, hb2,
           sW1, sb1, sW2, sb2, Wh, Ws, P1, pb1, P2, pb2, edge_index):
    pad = _EP - _E
    srcp = jnp.concatenate(
        [edge_index[0], (jnp.arange(pad, dtype=jnp.int32) % _N)])
    dstp = jnp.concatenate(
        [edge_index[1],
         _N + (jnp.arange(pad, dtype=jnp.int32) % (_NP - _N))])
    src = srcp.reshape(_NW, _NGRP, _GC, _C)
    dst = dstp.reshape(_NW, _NGRP, _GC, _C)
    zeros_d = jnp.zeros((_NP, _D), jnp.float32)

    def agg(h, s3, d3, z):
        return _sc_agg(_D)(h, s3, d3, z)[:, :_N]

    degp = _sc_deg()(dstp.reshape(_NW, _NCHUNK, _C))
    degn = _tc_degsum(degp).reshape(_DROW * _D, 1)[:_N]

    hE, hH, hS = _tc1(tokens, eW1, eb1.reshape(1, -1), eW2, eb2.reshape(1, -1),
                      hW1, hb1.reshape(1, -1), hW2, hb2.reshape(1, -1),
                      sW1, sb1.reshape(1, -1), sW2, sb2.reshape(1, -1))
    aE = agg(hE, src, dst, zeros_d)
    aH = agg(hH, src, dst, zeros_d)
    aS = agg(hS, src, dst, zeros_d)
    xE, xH, xS, u, dinv = _tc2(hE, hH, hS, aE, aH, aS, degn, Wh[0])

    for i in range(Wh.shape[0]):
        aU = agg(u, src, dst, zeros_d)
        xH, v = _tc3(xS, u, aU, dinv, Ws[i][:_D], Ws[i][_D:])
        aV = agg(v, src, dst, zeros_d)
        WhN = Wh[(i + 1) % Wh.shape[0]]
        xE, xS, u = _tc4(xE, xH, v, aV, dinv,
                         P1[i][:_D], P1[i][_D:2 * _D], P1[i][2 * _D:],
                         pb1[i].reshape(1, -1), P2[i], pb2[i].reshape(1, -1),
                         WhN)
    return (xE, xH, xS)


# speedup vs baseline: 1.0295x; 1.0295x over previous
import functools

import jax
import jax.numpy as jnp
from jax import lax
from jax.experimental import pallas as pl
from jax.experimental.pallas import tpu as pltpu
from jax.experimental.pallas import tpu_sc as plsc

_N = 10000
_E = 320000
_D = 128
_HID = 256

_NC = 2
_NS = 16
_NW = _NC * _NS
_C = 128
_EP = 327680
_EW = _EP // _NW
_NCHUNK = _EW // _C
_GC = 5
_NGRP = _NCHUNK // _GC
_NBUF = 2
_NP = 10112
_RPT = _NP // _NS
_DEGW = 128

_BLK = 1000
_GRID = _N // _BLK


@functools.cache
def _sc_agg(width):
    mesh = plsc.VectorSubcoreMesh(core_axis_name="c", subcore_axis_name="s")

    def body(h_hbm, src_hbm, dst_hbm, zero_hbm, out_hbm,
             srcb, dstb, *bufs_and_sems):
        rows = bufs_and_sems[:_NBUF]
        acc_sh = bufs_and_sems[_NBUF]
        gs = bufs_and_sems[_NBUF + 1:2 * _NBUF + 1]
        ss = bufs_and_sems[2 * _NBUF + 1:3 * _NBUF + 1]
        isem = bufs_and_sems[3 * _NBUF + 1]
        c = lax.axis_index("c")
        s = lax.axis_index("s")
        wid = s * _NC + c
        sw = src_hbm.at[wid]
        dw = dst_hbm.at[wid]
        pltpu.sync_copy(sw.at[0], srcb.at[0])
        pltpu.sync_copy(dw.at[0], dstb.at[0])
        pltpu.sync_copy(zero_hbm.at[pl.ds(s * _RPT, _RPT)],
                        acc_sh.at[pl.ds(s * _RPT, _RPT)])
        plsc.subcore_barrier()

        def grp(g, carry):
            b = g % 2
            nb = (g + 1) % 2

            @pl.when(g < _NGRP - 1)
            def _prefetch():
                pltpu.async_copy(sw.at[g + 1], srcb.at[nb], isem)
                pltpu.async_copy(dw.at[g + 1], dstb.at[nb], isem)

            S = srcb.at[b]
            Dx = dstb.at[b]
            for j in range(min(_NBUF, _GC)):
                pltpu.async_copy(h_hbm.at[S.at[j]], rows[j % _NBUF],
                                 gs[j % _NBUF])
            for j in range(_GC):
                p = j % _NBUF
                pltpu.make_async_copy(h_hbm.at[S.at[j]], rows[p],
                                      gs[p]).wait()
                pltpu.async_copy(rows[p], acc_sh.at[Dx.at[j]], ss[p],
                                 add=True)
                if j + _NBUF < _GC:
                    pltpu.make_async_copy(rows[p], acc_sh.at[Dx.at[j]],
                                          ss[p]).wait()
                    pltpu.async_copy(h_hbm.at[S.at[j + _NBUF]], rows[p],
                                     gs[p])
            for j in range(max(0, _GC - _NBUF), _GC):
                pltpu.make_async_copy(rows[j % _NBUF], acc_sh.at[Dx.at[j]],
                                      ss[j % _NBUF]).wait()

            @pl.when(g < _NGRP - 1)
            def _prefetch_wait():
                pltpu.make_async_copy(sw.at[g + 1], srcb.at[nb],
                                      isem).wait()
                pltpu.make_async_copy(dw.at[g + 1], dstb.at[nb],
                                      isem).wait()

            return carry

        lax.fori_loop(0, _NGRP, grp, 0)
        plsc.subcore_barrier()
        pltpu.sync_copy(acc_sh.at[pl.ds(s * _RPT, _RPT)],
                        out_hbm.at[c].at[pl.ds(s * _RPT, _RPT)])

    return pl.kernel(
        body,
        out_type=jax.ShapeDtypeStruct((_NC, _NP, width), jnp.float32),
        mesh=mesh,
        scratch_types=(
            [pltpu.VMEM((2, _GC, _C), jnp.int32),
             pltpu.VMEM((2, _GC, _C), jnp.int32)]
            + [pltpu.VMEM((_C, width), jnp.float32)] * _NBUF
            + [pltpu.VMEM_SHARED((_NP, width), jnp.float32)]
            + [pltpu.SemaphoreType.DMA] * (2 * _NBUF + 1)
        ),
    )


_DROW = 80


@functools.cache
def _sc_deg():
    mesh = plsc.VectorSubcoreMesh(core_axis_name="c", subcore_axis_name="s")

    nwords = _DROW * _D

    def body(dst_hbm, out_hbm, dst_v, cnt_v):
        c = lax.axis_index("c")
        s = lax.axis_index("s")
        wid = s * _NC + c
        pltpu.sync_copy(dst_hbm.at[wid], dst_v)
        zero16 = jnp.zeros((16,), jnp.float32)

        def zero(i, carry):
            cnt_v[pl.ds(16 * i, 16)] = zero16
            return carry

        lax.fori_loop(0, nwords // 16, zero, 0)
        ones16 = jnp.full((16,), 1.0, jnp.float32)

        def chunk(k, carry):
            for m in range(_C // 16):
                idx = dst_v[k, pl.ds(16 * m, 16)]
                plsc.addupdate_scatter(cnt_v, [idx], ones16)
            return carry

        lax.fori_loop(0, _NCHUNK, chunk, 0)
        pltpu.sync_copy(cnt_v, out_hbm.at[pl.ds(wid * nwords, nwords)])

    return pl.kernel(
        body,
        out_type=jax.ShapeDtypeStruct((_NW * nwords,), jnp.float32),
        mesh=mesh,
        compiler_params=pltpu.CompilerParams(needs_layout_passes=False),
        scratch_types=[
            pltpu.VMEM((_NCHUNK, _C), jnp.int32),
            pltpu.VMEM((nwords,), jnp.float32),
        ],
    )


def _tc_degsum_body(dp, out):
    out[...] = jnp.sum(dp[...], axis=0)


def _tc_degsum(dp):
    return pl.pallas_call(
        _tc_degsum_body,
        grid=(1,),
        in_specs=[pl.BlockSpec((_NW, _DROW, _D), lambda i: (0, 0, 0))],
        out_specs=pl.BlockSpec((_DROW, _D), lambda i: (0, 0)),
        out_shape=jax.ShapeDtypeStruct((_DROW, _D), jnp.float32),
    )(dp.reshape(_NW, _DROW, _D))


def _lane0(shape):
    return lax.broadcasted_iota(jnp.int32, shape, 1) == 0


def _col0(a):
    return jnp.sum(jnp.where(_lane0(a.shape), a, 0.0), axis=-1, keepdims=True)


def _tailnorm(a):
    sq = jnp.where(_lane0(a.shape), 0.0, a * a)
    return jnp.sqrt(jnp.sum(sq, axis=-1, keepdims=True))


def _dot(a, b):
    return jnp.dot(a, b, preferred_element_type=jnp.float32)


def _l_expmap0(u):
    un = jnp.clip(_tailnorm(u), 1e-7, 5.0)
    e = jnp.exp(un)
    ei = 1.0 / e
    ch = 0.5 * (e + ei)
    sh = 0.5 * (e - ei)
    return jnp.where(_lane0(u.shape), ch, u * (sh / un))


def _l_logmap0(xx):
    x0 = jnp.maximum(_col0(xx), 1.0 + 1e-7)
    d = jnp.log(x0 + jnp.sqrt(x0 * x0 - 1.0))
    nr = jnp.maximum(_tailnorm(xx), 1e-7)
    return jnp.where(_lane0(xx.shape), 0.0, xx * (d / nr))


def _s_expmap0(u):
    import numpy as _np
    un = jnp.clip(_tailnorm(u), 1e-7, _np.pi - 1e-2)
    return jnp.where(_lane0(u.shape), jnp.cos(un), u * (jnp.sin(un) / un))


def _arccos(x):
    import numpy as _np
    ax = jnp.abs(x)
    small = ax <= 0.5
    z = jnp.where(small, x * x, 0.5 * (1.0 - ax))
    s = jnp.where(small, ax, jnp.sqrt(z))
    r = (z * (0.16666586697 + z * (-0.042743422091 + z * (-0.0086563630030)))
         / (1.0 + z * (-0.70662963390)))
    asn = s + s * r
    sgn_asn = jnp.where(x < 0.0, -asn, asn)
    small_th = jnp.float32(_np.pi / 2) - sgn_asn
    big_th = jnp.where(x < 0.0, jnp.float32(_np.pi) - 2.0 * asn, 2.0 * asn)
    return jnp.where(small, small_th, big_th)


def _s_logmap0(xx):
    x0 = jnp.clip(_col0(xx), -1.0 + 1e-6, 1.0 - 1e-6)
    th = _arccos(x0)
    nr = jnp.maximum(_tailnorm(xx), 1e-7)
    return jnp.where(_lane0(xx.shape), 0.0, xx * (th / nr))


def _l_proju(x, v):
    inner = jnp.sum(jnp.where(_lane0(x.shape), -x * v, x * v),
                    axis=-1, keepdims=True)
    return v + inner * x


def _l_transp0back(x, v):
    x0 = _col0(x)
    v0 = _col0(v)
    xo = x + jnp.where(_lane0(x.shape), 1.0, 0.0)
    return v + (v0 / (1.0 + x0)) * xo


def _s_proju(x, v):
    return v - jnp.sum(x * v, axis=-1, keepdims=True) * x


def _s_transp0back(x, v):
    x0 = _col0(x)
    v0 = _col0(v)
    xo = x + jnp.where(_lane0(x.shape), 1.0, 0.0)
    return v - (v0 / jnp.maximum(1.0 + x0, 1e-5)) * xo


def _mlp(t, W1, b1, W2, b2):
    return _dot(jnp.maximum(_dot(t, W1) + b1, 0.0), W2) + b2


def _node_spec():
    return pl.BlockSpec((_BLK, _D), lambda i: (i, 0))


def _pair_spec(width):
    return pl.BlockSpec((_NC, _BLK, width), lambda i: (0, i, 0))


def _full_spec(shape):
    nd = len(shape)
    return pl.BlockSpec(shape, lambda i: (0,) * nd)


def _tc1_body(tok, eW1, eb1, eW2, eb2, hW1, hb1, hW2, hb2,
              sW1, sb1, sW2, sb2, hE_o, hH_o, hS_o):
    t = tok[...]
    hE_o[...] = _mlp(t, eW1[...], eb1[...], eW2[...], eb2[...])
    hH_o[...] = _mlp(t, hW1[...], hb1[...], hW2[...], hb2[...])
    hS_o[...] = _mlp(t, sW1[...], sb1[...], sW2[...], sb2[...])


def _tc1(tok, eW1, eb1, eW2, eb2, hW1, hb1, hW2, hb2, sW1, sb1, sW2, sb2):
    ws = [eW1, eb1, eW2, eb2, hW1, hb1, hW2, hb2, sW1, sb1, sW2, sb2]
    return pl.pallas_call(
        _tc1_body,
        grid=(_GRID,),
        in_specs=[_node_spec()] + [_full_spec(w.shape) for w in ws],
        out_specs=[_node_spec()] * 3,
        out_shape=[jax.ShapeDtypeStruct((_N, _D), jnp.float32)] * 3,
    )(tok, *ws)


def _tc2_body(hE, hH, hS, aE, aH, aS, degn, Wh0,
              xE_o, xH_o, xS_o, u_o, dinv_o):
    dinv = 1.0 / (1.0 + degn[...])
    dinv_o[...] = jnp.broadcast_to(dinv, (_BLK, _D))
    xE_o[...] = (hE[...] + aE[...][0] + aE[...][1]) * dinv
    xH = _l_expmap0((hH[...] + aH[...][0] + aH[...][1]) * dinv)
    xH_o[...] = xH
    xS_o[...] = _s_expmap0((hS[...] + aS[...][0] + aS[...][1]) * dinv)
    u_o[...] = _dot(_l_logmap0(xH), Wh0[...])


def _tc2(hE, hH, hS, aE, aH, aS, degn, Wh0):
    return pl.pallas_call(
        _tc2_body,
        grid=(_GRID,),
        in_specs=[_node_spec()] * 3 + [_pair_spec(_D)] * 3
        + [pl.BlockSpec((_BLK, 1), lambda i: (i, 0)), _full_spec(Wh0.shape)],
        out_specs=[_node_spec()] * 5,
        out_shape=[jax.ShapeDtypeStruct((_N, _D), jnp.float32)] * 5,
    )(hE, hH, hS, aE, aH, aS, degn, Wh0)


def _tc3_body(xS, u, aU, dinv, WsA, WsB, xH_o, v_o):
    xH = _l_expmap0((u[...] + aU[...][0] + aU[...][1]) * dinv[...])
    xH_o[...] = xH
    v_o[...] = (_dot(_s_logmap0(xS[...]), WsA[...])
                + _dot(_l_logmap0(xH), WsB[...]))


def _tc3(xS, u, aU, dinv, WsA, WsB):
    return pl.pallas_call(
        _tc3_body,
        grid=(_GRID,),
        in_specs=[_node_spec(), _node_spec(), _pair_spec(_D), _node_spec(),
                  _full_spec(WsA.shape), _full_spec(WsB.shape)],
        out_specs=[_node_spec()] * 2,
        out_shape=[jax.ShapeDtypeStruct((_N, _D), jnp.float32)] * 2,
    )(xS, u, aU, dinv, WsA, WsB)


def _tc4_body(xE, xH, v, aV, dinv, P1a, P1b, P1c, pb1, P2, pb2, WhN,
              xE_o, xS_o, u_o):
    xS = _s_expmap0((v[...] + aV[...][0] + aV[...][1]) * dinv[...])
    xS_o[...] = xS
    xEi = xE[...]
    xHi = xH[...]
    HE = _l_transp0back(xHi, _l_proju(xHi, xEi))
    SE = _s_transp0back(xS, _s_proju(xS, xEi))
    hmid = jnp.maximum(_dot(xEi, P1a[...]) + _dot(HE, P1b[...])
                       + _dot(SE, P1c[...]) + pb1[...], 0.0)
    xE_o[...] = _dot(hmid, P2[...]) + pb2[...]
    u_o[...] = _dot(_l_logmap0(xHi), WhN[...])


def _tc4(xE, xH, v, aV, dinv, P1a, P1b, P1c, pb1, P2, pb2, WhN):
    ws = [P1a, P1b, P1c, pb1, P2, pb2, WhN]
    return pl.pallas_call(
        _tc4_body,
        grid=(_GRID,),
        in_specs=[_node_spec(), _node_spec(), _node_spec(), _pair_spec(_D),
                  _node_spec()] + [_full_spec(w.shape) for w in ws],
        out_specs=[_node_spec()] * 3,
        out_shape=[jax.ShapeDtypeStruct((_N, _D), jnp.float32)] * 3,
    )(xE, xH, v, aV, dinv, *ws)


def kernel(x, tokens, eW1, eb1, eW2, eb2, hW1, hb1, hW2, hb2,
           sW1, sb1, sW2, sb2, Wh, Ws, P1, pb1, P2, pb2, edge_index):
    pad = _EP - _E
    srcp = jnp.concatenate(
        [edge_index[0], (jnp.arange(pad, dtype=jnp.int32) % _N)])
    dstp = jnp.concatenate(
        [edge_index[1],
         _N + (jnp.arange(pad, dtype=jnp.int32) % (_NP - _N))])
    src = srcp.reshape(_NW, _NGRP, _GC, _C)
    dst = dstp.reshape(_NW, _NGRP, _GC, _C)
    zeros_d = jnp.zeros((_NP, _D), jnp.float32)

    def agg(h, s3, d3, z):
        return _sc_agg(_D)(h, s3, d3, z)[:, :_N]

    degp = _sc_deg()(dstp.reshape(_NW, _NCHUNK, _C))
    degn = _tc_degsum(degp).reshape(_DROW * _D, 1)[:_N]

    hE, hH, hS = _tc1(tokens, eW1, eb1.reshape(1, -1), eW2, eb2.reshape(1, -1),
                      hW1, hb1.reshape(1, -1), hW2, hb2.reshape(1, -1),
                      sW1, sb1.reshape(1, -1), sW2, sb2.reshape(1, -1))
    aE = agg(hE, src, dst, zeros_d)
    aH = agg(hH, src, dst, zeros_d)
    aS = agg(hS, src, dst, zeros_d)
    xE, xH, xS, u, dinv = _tc2(hE, hH, hS, aE, aH, aS, degn, Wh[0])

    for i in range(Wh.shape[0]):
        aU = agg(u, src, dst, zeros_d)
        xH, v = _tc3(xS, u, aU, dinv, Ws[i][:_D], Ws[i][_D:])
        aV = agg(v, src, dst, zeros_d)
        WhN = Wh[(i + 1) % Wh.shape[0]]
        xE, xS, u = _tc4(xE, xH, v, aV, dinv,
                         P1[i][:_D], P1[i][_D:2 * _D], P1[i][2 * _D:],
                         pb1[i].reshape(1, -1), P2[i], pb2[i].reshape(1, -1),
                         WhN)
    return (xE, xH, xS)


# speedup vs baseline: 1.0398x; 1.0100x over previous
import functools

import jax
import jax.numpy as jnp
from jax import lax
from jax.experimental import pallas as pl
from jax.experimental.pallas import tpu as pltpu
from jax.experimental.pallas import tpu_sc as plsc

_N = 10000
_E = 320000
_D = 128
_HID = 256

_NC = 2
_NS = 16
_NW = _NC * _NS
_C = 128
_EP = 327680
_EW = _EP // _NW
_NCHUNK = _EW // _C
_GC = 5
_NGRP = _NCHUNK // _GC
_NBUF = 2
_NP = 10112
_RPT = _NP // _NS
_DEGW = 128

_BLK = 1000
_GRID = _N // _BLK


def _agg_pass(h_hbm, out2, sw, dw, zero_hbm, srcb, dstb, rows, acc_sh,
              gs, ss, isem, c, s):
    pltpu.sync_copy(sw.at[0], srcb.at[0])
    pltpu.sync_copy(dw.at[0], dstb.at[0])
    pltpu.sync_copy(zero_hbm.at[pl.ds(s * _RPT, _RPT)],
                    acc_sh.at[pl.ds(s * _RPT, _RPT)])
    plsc.subcore_barrier()

    def grp(g, carry):
        b = g % 2
        nb = (g + 1) % 2

        @pl.when(g < _NGRP - 1)
        def _prefetch():
            pltpu.async_copy(sw.at[g + 1], srcb.at[nb], isem)
            pltpu.async_copy(dw.at[g + 1], dstb.at[nb], isem)

        S = srcb.at[b]
        Dx = dstb.at[b]
        for j in range(min(_NBUF, _GC)):
            pltpu.async_copy(h_hbm.at[S.at[j]], rows[j % _NBUF],
                             gs[j % _NBUF])
        for j in range(_GC):
            p = j % _NBUF
            pltpu.make_async_copy(h_hbm.at[S.at[j]], rows[p], gs[p]).wait()
            pltpu.async_copy(rows[p], acc_sh.at[Dx.at[j]], ss[p], add=True)
            if j + _NBUF < _GC:
                pltpu.make_async_copy(rows[p], acc_sh.at[Dx.at[j]],
                                      ss[p]).wait()
                pltpu.async_copy(h_hbm.at[S.at[j + _NBUF]], rows[p], gs[p])
        for j in range(max(0, _GC - _NBUF), _GC):
            pltpu.make_async_copy(rows[j % _NBUF], acc_sh.at[Dx.at[j]],
                                  ss[j % _NBUF]).wait()

        @pl.when(g < _NGRP - 1)
        def _prefetch_wait():
            pltpu.make_async_copy(sw.at[g + 1], srcb.at[nb], isem).wait()
            pltpu.make_async_copy(dw.at[g + 1], dstb.at[nb], isem).wait()

        return carry

    lax.fori_loop(0, _NGRP, grp, 0)
    plsc.subcore_barrier()
    pltpu.sync_copy(acc_sh.at[pl.ds(s * _RPT, _RPT)],
                    out2.at[c].at[pl.ds(s * _RPT, _RPT)])


def _agg_scratch(width):
    return (
        [pltpu.VMEM((2, _GC, _C), jnp.int32),
         pltpu.VMEM((2, _GC, _C), jnp.int32)]
        + [pltpu.VMEM((_C, width), jnp.float32)] * _NBUF
        + [pltpu.VMEM_SHARED((_NP, width), jnp.float32)]
        + [pltpu.SemaphoreType.DMA] * (2 * _NBUF + 1)
    )


def _split_bufs(bufs_and_sems):
    rows = bufs_and_sems[:_NBUF]
    acc_sh = bufs_and_sems[_NBUF]
    gs = bufs_and_sems[_NBUF + 1:2 * _NBUF + 1]
    ss = bufs_and_sems[2 * _NBUF + 1:3 * _NBUF + 1]
    isem = bufs_and_sems[3 * _NBUF + 1]
    return rows, acc_sh, gs, ss, isem


@functools.cache
def _sc_agg(width):
    mesh = plsc.VectorSubcoreMesh(core_axis_name="c", subcore_axis_name="s")

    def body(h_hbm, src_hbm, dst_hbm, zero_hbm, out_hbm,
             srcb, dstb, *bufs_and_sems):
        rows, acc_sh, gs, ss, isem = _split_bufs(bufs_and_sems)
        c = lax.axis_index("c")
        s = lax.axis_index("s")
        wid = s * _NC + c
        _agg_pass(h_hbm, out_hbm, src_hbm.at[wid], dst_hbm.at[wid],
                  zero_hbm, srcb, dstb, rows, acc_sh, gs, ss, isem, c, s)

    return pl.kernel(
        body,
        out_type=jax.ShapeDtypeStruct((_NC, _NP, width), jnp.float32),
        mesh=mesh,
        scratch_types=_agg_scratch(width),
    )


@functools.cache
def _sc_agg3():
    mesh = plsc.VectorSubcoreMesh(core_axis_name="c", subcore_axis_name="s")

    def body(h0, h1, h2, src_hbm, dst_hbm, zero_hbm, out_hbm,
             srcb, dstb, *bufs_and_sems):
        rows, acc_sh, gs, ss, isem = _split_bufs(bufs_and_sems)
        c = lax.axis_index("c")
        s = lax.axis_index("s")
        wid = s * _NC + c
        for t, h in enumerate((h0, h1, h2)):
            _agg_pass(h, out_hbm.at[t], src_hbm.at[wid], dst_hbm.at[wid],
                      zero_hbm, srcb, dstb, rows, acc_sh, gs, ss, isem, c, s)

    return pl.kernel(
        body,
        out_type=jax.ShapeDtypeStruct((3, _NC, _NP, _D), jnp.float32),
        mesh=mesh,
        scratch_types=_agg_scratch(_D),
    )


_DROW = 80


@functools.cache
def _sc_deg():
    mesh = plsc.VectorSubcoreMesh(core_axis_name="c", subcore_axis_name="s")

    nwords = _DROW * _D

    def body(dst_hbm, out_hbm, dst_v, cnt_v):
        c = lax.axis_index("c")
        s = lax.axis_index("s")
        wid = s * _NC + c
        pltpu.sync_copy(dst_hbm.at[wid], dst_v)
        zero16 = jnp.zeros((16,), jnp.float32)

        def zero(i, carry):
            cnt_v[pl.ds(16 * i, 16)] = zero16
            return carry

        lax.fori_loop(0, nwords // 16, zero, 0)
        ones16 = jnp.full((16,), 1.0, jnp.float32)

        def chunk(k, carry):
            for m in range(_C // 16):
                idx = dst_v[k, pl.ds(16 * m, 16)]
                plsc.addupdate_scatter(cnt_v, [idx], ones16)
            return carry

        lax.fori_loop(0, _NCHUNK, chunk, 0)
        pltpu.sync_copy(cnt_v, out_hbm.at[pl.ds(wid * nwords, nwords)])

    return pl.kernel(
        body,
        out_type=jax.ShapeDtypeStruct((_NW * nwords,), jnp.float32),
        mesh=mesh,
        compiler_params=pltpu.CompilerParams(needs_layout_passes=False),
        scratch_types=[
            pltpu.VMEM((_NCHUNK, _C), jnp.int32),
            pltpu.VMEM((nwords,), jnp.float32),
        ],
    )


def _tc_degsum_body(dp, out):
    out[...] = jnp.sum(dp[...], axis=0)


def _tc_degsum(dp):
    return pl.pallas_call(
        _tc_degsum_body,
        grid=(1,),
        in_specs=[pl.BlockSpec((_NW, _DROW, _D), lambda i: (0, 0, 0))],
        out_specs=pl.BlockSpec((_DROW, _D), lambda i: (0, 0)),
        out_shape=jax.ShapeDtypeStruct((_DROW, _D), jnp.float32),
    )(dp.reshape(_NW, _DROW, _D))


def _lane0(shape):
    return lax.broadcasted_iota(jnp.int32, shape, 1) == 0


def _col0(a):
    return jnp.sum(jnp.where(_lane0(a.shape), a, 0.0), axis=-1, keepdims=True)


def _tailnorm(a):
    sq = jnp.where(_lane0(a.shape), 0.0, a * a)
    return jnp.sqrt(jnp.sum(sq, axis=-1, keepdims=True))


def _dot(a, b):
    return jnp.dot(a, b, preferred_element_type=jnp.float32)


def _l_expmap0(u):
    un = jnp.clip(_tailnorm(u), 1e-7, 5.0)
    e = jnp.exp(un)
    ei = 1.0 / e
    ch = 0.5 * (e + ei)
    sh = 0.5 * (e - ei)
    return jnp.where(_lane0(u.shape), ch, u * (sh / un))


def _l_logmap0(xx):
    x0 = jnp.maximum(_col0(xx), 1.0 + 1e-7)
    d = jnp.log(x0 + jnp.sqrt(x0 * x0 - 1.0))
    nr = jnp.maximum(_tailnorm(xx), 1e-7)
    return jnp.where(_lane0(xx.shape), 0.0, xx * (d / nr))


def _s_expmap0(u):
    import numpy as _np
    un = jnp.clip(_tailnorm(u), 1e-7, _np.pi - 1e-2)
    return jnp.where(_lane0(u.shape), jnp.cos(un), u * (jnp.sin(un) / un))


def _arccos(x):
    import numpy as _np
    ax = jnp.abs(x)
    small = ax <= 0.5
    z = jnp.where(small, x * x, 0.5 * (1.0 - ax))
    s = jnp.where(small, ax, jnp.sqrt(z))
    r = (z * (0.16666586697 + z * (-0.042743422091 + z * (-0.0086563630030)))
         / (1.0 + z * (-0.70662963390)))
    asn = s + s * r
    sgn_asn = jnp.where(x < 0.0, -asn, asn)
    small_th = jnp.float32(_np.pi / 2) - sgn_asn
    big_th = jnp.where(x < 0.0, jnp.float32(_np.pi) - 2.0 * asn, 2.0 * asn)
    return jnp.where(small, small_th, big_th)


def _s_logmap0(xx):
    x0 = jnp.clip(_col0(xx), -1.0 + 1e-6, 1.0 - 1e-6)
    th = _arccos(x0)
    nr = jnp.maximum(_tailnorm(xx), 1e-7)
    return jnp.where(_lane0(xx.shape), 0.0, xx * (th / nr))


def _l_proju(x, v):
    inner = jnp.sum(jnp.where(_lane0(x.shape), -x * v, x * v),
                    axis=-1, keepdims=True)
    return v + inner * x


def _l_transp0back(x, v):
    x0 = _col0(x)
    v0 = _col0(v)
    xo = x + jnp.where(_lane0(x.shape), 1.0, 0.0)
    return v + (v0 / (1.0 + x0)) * xo


def _s_proju(x, v):
    return v - jnp.sum(x * v, axis=-1, keepdims=True) * x


def _s_transp0back(x, v):
    x0 = _col0(x)
    v0 = _col0(v)
    xo = x + jnp.where(_lane0(x.shape), 1.0, 0.0)
    return v - (v0 / jnp.maximum(1.0 + x0, 1e-5)) * xo


def _mlp(t, W1, b1, W2, b2):
    return _dot(jnp.maximum(_dot(t, W1) + b1, 0.0), W2) + b2


def _node_spec():
    return pl.BlockSpec((_BLK, _D), lambda i: (i, 0))


def _pair_spec(width):
    return pl.BlockSpec((_NC, _BLK, width), lambda i: (0, i, 0))


def _full_spec(shape):
    nd = len(shape)
    return pl.BlockSpec(shape, lambda i: (0,) * nd)


def _tc1_body(tok, eW1, eb1, eW2, eb2, hW1, hb1, hW2, hb2,
              sW1, sb1, sW2, sb2, hE_o, hH_o, hS_o):
    t = tok[...]
    hE_o[...] = _mlp(t, eW1[...], eb1[...], eW2[...], eb2[...])
    hH_o[...] = _mlp(t, hW1[...], hb1[...], hW2[...], hb2[...])
    hS_o[...] = _mlp(t, sW1[...], sb1[...], sW2[...], sb2[...])


def _tc1(tok, eW1, eb1, eW2, eb2, hW1, hb1, hW2, hb2, sW1, sb1, sW2, sb2):
    ws = [eW1, eb1, eW2, eb2, hW1, hb1, hW2, hb2, sW1, sb1, sW2, sb2]
    return pl.pallas_call(
        _tc1_body,
        grid=(_GRID,),
        in_specs=[_node_spec()] + [_full_spec(w.shape) for w in ws],
        out_specs=[_node_spec()] * 3,
        out_shape=[jax.ShapeDtypeStruct((_N, _D), jnp.float32)] * 3,
    )(tok, *ws)


def _tc2_body(hE, hH, hS, aE, aH, aS, degn, Wh0,
              xE_o, xH_o, xS_o, u_o, dinv_o):
    dinv = 1.0 / (1.0 + degn[...])
    dinv_o[...] = jnp.broadcast_to(dinv, (_BLK, _D))
    xE_o[...] = (hE[...] + aE[...][0] + aE[...][1]) * dinv
    xH = _l_expmap0((hH[...] + aH[...][0] + aH[...][1]) * dinv)
    xH_o[...] = xH
    xS_o[...] = _s_expmap0((hS[...] + aS[...][0] + aS[...][1]) * dinv)
    u_o[...] = _dot(_l_logmap0(xH), Wh0[...])


def _tc2(hE, hH, hS, aE, aH, aS, degn, Wh0):
    return pl.pallas_call(
        _tc2_body,
        grid=(_GRID,),
        in_specs=[_node_spec()] * 3 + [_pair_spec(_D)] * 3
        + [pl.BlockSpec((_BLK, 1), lambda i: (i, 0)), _full_spec(Wh0.shape)],
        out_specs=[_node_spec()] * 5,
        out_shape=[jax.ShapeDtypeStruct((_N, _D), jnp.float32)] * 5,
    )(hE, hH, hS, aE, aH, aS, degn, Wh0)


def _tc3_body(xS, u, aU, dinv, WsA, WsB, xH_o, v_o):
    xH = _l_expmap0((u[...] + aU[...][0] + aU[...][1]) * dinv[...])
    xH_o[...] = xH
    v_o[...] = (_dot(_s_logmap0(xS[...]), WsA[...])
                + _dot(_l_logmap0(xH), WsB[...]))


def _tc3(xS, u, aU, dinv, WsA, WsB):
    return pl.pallas_call(
        _tc3_body,
        grid=(_GRID,),
        in_specs=[_node_spec(), _node_spec(), _pair_spec(_D), _node_spec(),
                  _full_spec(WsA.shape), _full_spec(WsB.shape)],
        out_specs=[_node_spec()] * 2,
        out_shape=[jax.ShapeDtypeStruct((_N, _D), jnp.float32)] * 2,
    )(xS, u, aU, dinv, WsA, WsB)


def _tc4_body(xE, xH, v, aV, dinv, P1a, P1b, P1c, pb1, P2, pb2, WhN,
              xE_o, xS_o, u_o):
    xS = _s_expmap0((v[...] + aV[...][0] + aV[...][1]) * dinv[...])
    xS_o[...] = xS
    xEi = xE[...]
    xHi = xH[...]
    HE = _l_transp0back(xHi, _l_proju(xHi, xEi))
    SE = _s_transp0back(xS, _s_proju(xS, xEi))
    hmid = jnp.maximum(_dot(xEi, P1a[...]) + _dot(HE, P1b[...])
                       + _dot(SE, P1c[...]) + pb1[...], 0.0)
    xE_o[...] = _dot(hmid, P2[...]) + pb2[...]
    u_o[...] = _dot(_l_logmap0(xHi), WhN[...])


def _tc4(xE, xH, v, aV, dinv, P1a, P1b, P1c, pb1, P2, pb2, WhN):
    ws = [P1a, P1b, P1c, pb1, P2, pb2, WhN]
    return pl.pallas_call(
        _tc4_body,
        grid=(_GRID,),
        in_specs=[_node_spec(), _node_spec(), _node_spec(), _pair_spec(_D),
                  _node_spec()] + [_full_spec(w.shape) for w in ws],
        out_specs=[_node_spec()] * 3,
        out_shape=[jax.ShapeDtypeStruct((_N, _D), jnp.float32)] * 3,
    )(xE, xH, v, aV, dinv, *ws)


def kernel(x, tokens, eW1, eb1, eW2, eb2, hW1, hb1, hW2, hb2,
           sW1, sb1, sW2, sb2, Wh, Ws, P1, pb1, P2, pb2, edge_index):
    pad = _EP - _E
    srcp = jnp.concatenate(
        [edge_index[0], (jnp.arange(pad, dtype=jnp.int32) % _N)])
    dstp = jnp.concatenate(
        [edge_index[1],
         _N + (jnp.arange(pad, dtype=jnp.int32) % (_NP - _N))])
    src = srcp.reshape(_NW, _NGRP, _GC, _C)
    dst = dstp.reshape(_NW, _NGRP, _GC, _C)
    zeros_d = jnp.zeros((_NP, _D), jnp.float32)

    def agg(h, s3, d3, z):
        return _sc_agg(_D)(h, s3, d3, z)[:, :_N]

    degp = _sc_deg()(dstp.reshape(_NW, _NCHUNK, _C))
    degn = _tc_degsum(degp).reshape(_DROW * _D, 1)[:_N]

    hE, hH, hS = _tc1(tokens, eW1, eb1.reshape(1, -1), eW2, eb2.reshape(1, -1),
                      hW1, hb1.reshape(1, -1), hW2, hb2.reshape(1, -1),
                      sW1, sb1.reshape(1, -1), sW2, sb2.reshape(1, -1))
    a3 = _sc_agg3()(hE, hH, hS, src, dst, zeros_d)
    aE = a3[0, :, :_N]
    aH = a3[1, :, :_N]
    aS = a3[2, :, :_N]
    xE, xH, xS, u, dinv = _tc2(hE, hH, hS, aE, aH, aS, degn, Wh[0])

    for i in range(Wh.shape[0]):
        aU = agg(u, src, dst, zeros_d)
        xH, v = _tc3(xS, u, aU, dinv, Ws[i][:_D], Ws[i][_D:])
        aV = agg(v, src, dst, zeros_d)
        WhN = Wh[(i + 1) % Wh.shape[0]]
        xE, xS, u = _tc4(xE, xH, v, aV, dinv,
                         P1[i][:_D], P1[i][_D:2 * _D], P1[i][2 * _D:],
                         pb1[i].reshape(1, -1), P2[i], pb2[i].reshape(1, -1),
                         WhN)
    return (xE, xH, xS)


# speedup vs baseline: 1.1198x; 1.0770x over previous
import functools

import jax
import jax.numpy as jnp
from jax import lax
from jax.experimental import pallas as pl
from jax.experimental.pallas import tpu as pltpu
from jax.experimental.pallas import tpu_sc as plsc

_N = 10000
_E = 320000
_D = 128
_HID = 256

_NC = 2
_NS = 16
_NW = _NC * _NS
_C = 128
_EP = 327680
_EW = _EP // _NW
_NCHUNK = _EW // _C
_GC = 8
_NGRP = _NCHUNK // _GC
_NBUF = 2
_NP = 10112
_RPT = _NP // _NS
_DEGW = 128

_BLK = 1000
_GRID = _N // _BLK


def _agg_pass(h_hbm, out2, sw, dw, zero_hbm, srcb, dstb, rows, acc_sh,
              gs, ss, isem, c, s):
    pltpu.sync_copy(sw.at[0], srcb.at[0])
    pltpu.sync_copy(dw.at[0], dstb.at[0])
    pltpu.sync_copy(zero_hbm.at[pl.ds(s * _RPT, _RPT)],
                    acc_sh.at[pl.ds(s * _RPT, _RPT)])
    plsc.subcore_barrier()

    def grp(g, carry):
        b = g % 2
        nb = (g + 1) % 2

        @pl.when(g < _NGRP - 1)
        def _prefetch():
            pltpu.async_copy(sw.at[g + 1], srcb.at[nb], isem)
            pltpu.async_copy(dw.at[g + 1], dstb.at[nb], isem)

        S = srcb.at[b]
        Dx = dstb.at[b]
        for j in range(min(_NBUF, _GC)):
            pltpu.async_copy(h_hbm.at[S.at[j]], rows[j % _NBUF],
                             gs[j % _NBUF])
        for j in range(_GC):
            p = j % _NBUF
            pltpu.make_async_copy(h_hbm.at[S.at[j]], rows[p], gs[p]).wait()
            pltpu.async_copy(rows[p], acc_sh.at[Dx.at[j]], ss[p], add=True)
            if j + _NBUF < _GC:
                pltpu.make_async_copy(rows[p], acc_sh.at[Dx.at[j]],
                                      ss[p]).wait()
                pltpu.async_copy(h_hbm.at[S.at[j + _NBUF]], rows[p], gs[p])
        for j in range(max(0, _GC - _NBUF), _GC):
            pltpu.make_async_copy(rows[j % _NBUF], acc_sh.at[Dx.at[j]],
                                  ss[j % _NBUF]).wait()

        @pl.when(g < _NGRP - 1)
        def _prefetch_wait():
            pltpu.make_async_copy(sw.at[g + 1], srcb.at[nb], isem).wait()
            pltpu.make_async_copy(dw.at[g + 1], dstb.at[nb], isem).wait()

        return carry

    lax.fori_loop(0, _NGRP, grp, 0)
    plsc.subcore_barrier()
    pltpu.sync_copy(acc_sh.at[pl.ds(s * _RPT, _RPT)],
                    out2.at[c].at[pl.ds(s * _RPT, _RPT)])


def _agg_scratch(width):
    return (
        [pltpu.VMEM((2, _GC, _C), jnp.int32),
         pltpu.VMEM((2, _GC, _C), jnp.int32)]
        + [pltpu.VMEM((_C, width), jnp.float32)] * _NBUF
        + [pltpu.VMEM_SHARED((_NP, width), jnp.float32)]
        + [pltpu.SemaphoreType.DMA] * (2 * _NBUF + 1)
    )


def _split_bufs(bufs_and_sems):
    rows = bufs_and_sems[:_NBUF]
    acc_sh = bufs_and_sems[_NBUF]
    gs = bufs_and_sems[_NBUF + 1:2 * _NBUF + 1]
    ss = bufs_and_sems[2 * _NBUF + 1:3 * _NBUF + 1]
    isem = bufs_and_sems[3 * _NBUF + 1]
    return rows, acc_sh, gs, ss, isem


@functools.cache
def _sc_agg(width):
    mesh = plsc.VectorSubcoreMesh(core_axis_name="c", subcore_axis_name="s")

    def body(h_hbm, src_hbm, dst_hbm, zero_hbm, out_hbm,
             srcb, dstb, *bufs_and_sems):
        rows, acc_sh, gs, ss, isem = _split_bufs(bufs_and_sems)
        c = lax.axis_index("c")
        s = lax.axis_index("s")
        wid = s * _NC + c
        _agg_pass(h_hbm, out_hbm, src_hbm.at[wid], dst_hbm.at[wid],
                  zero_hbm, srcb, dstb, rows, acc_sh, gs, ss, isem, c, s)

    return pl.kernel(
        body,
        out_type=jax.ShapeDtypeStruct((_NC, _NP, width), jnp.float32),
        mesh=mesh,
        scratch_types=_agg_scratch(width),
    )


@functools.cache
def _sc_agg3():
    mesh = plsc.VectorSubcoreMesh(core_axis_name="c", subcore_axis_name="s")

    def body(h0, h1, h2, src_hbm, dst_hbm, zero_hbm, out_hbm,
             srcb, dstb, *bufs_and_sems):
        rows, acc_sh, gs, ss, isem = _split_bufs(bufs_and_sems)
        c = lax.axis_index("c")
        s = lax.axis_index("s")
        wid = s * _NC + c
        for t, h in enumerate((h0, h1, h2)):
            _agg_pass(h, out_hbm.at[t], src_hbm.at[wid], dst_hbm.at[wid],
                      zero_hbm, srcb, dstb, rows, acc_sh, gs, ss, isem, c, s)

    return pl.kernel(
        body,
        out_type=jax.ShapeDtypeStruct((3, _NC, _NP, _D), jnp.float32),
        mesh=mesh,
        scratch_types=_agg_scratch(_D),
    )


_DROW = 80


@functools.cache
def _sc_deg():
    mesh = plsc.VectorSubcoreMesh(core_axis_name="c", subcore_axis_name="s")

    nwords = _DROW * _D

    def body(dst_hbm, out_hbm, dst_v, cnt_v):
        c = lax.axis_index("c")
        s = lax.axis_index("s")
        wid = s * _NC + c
        pltpu.sync_copy(dst_hbm.at[wid], dst_v)
        zero16 = jnp.zeros((16,), jnp.float32)

        def zero(i, carry):
            cnt_v[pl.ds(16 * i, 16)] = zero16
            return carry

        lax.fori_loop(0, nwords // 16, zero, 0)
        ones16 = jnp.full((16,), 1.0, jnp.float32)

        def chunk(k, carry):
            for m in range(_C // 16):
                idx = dst_v[k, pl.ds(16 * m, 16)]
                plsc.addupdate_scatter(cnt_v, [idx], ones16)
            return carry

        lax.fori_loop(0, _NCHUNK, chunk, 0)
        pltpu.sync_copy(cnt_v, out_hbm.at[pl.ds(wid * nwords, nwords)])

    return pl.kernel(
        body,
        out_type=jax.ShapeDtypeStruct((_NW * nwords,), jnp.float32),
        mesh=mesh,
        compiler_params=pltpu.CompilerParams(needs_layout_passes=False),
        scratch_types=[
            pltpu.VMEM((_NCHUNK, _C), jnp.int32),
            pltpu.VMEM((nwords,), jnp.float32),
        ],
    )


def _tc_degsum_body(dp, out):
    out[...] = jnp.sum(dp[...], axis=0)


def _tc_degsum(dp):
    return pl.pallas_call(
        _tc_degsum_body,
        grid=(1,),
        in_specs=[pl.BlockSpec((_NW, _DROW, _D), lambda i: (0, 0, 0))],
        out_specs=pl.BlockSpec((_DROW, _D), lambda i: (0, 0)),
        out_shape=jax.ShapeDtypeStruct((_DROW, _D), jnp.float32),
    )(dp.reshape(_NW, _DROW, _D))


def _lane0(shape):
    return lax.broadcasted_iota(jnp.int32, shape, 1) == 0


def _col0(a):
    return jnp.sum(jnp.where(_lane0(a.shape), a, 0.0), axis=-1, keepdims=True)


def _tailnorm(a):
    sq = jnp.where(_lane0(a.shape), 0.0, a * a)
    return jnp.sqrt(jnp.sum(sq, axis=-1, keepdims=True))


def _dot(a, b):
    return jnp.dot(a, b, preferred_element_type=jnp.float32)


def _l_expmap0(u):
    un = jnp.clip(_tailnorm(u), 1e-7, 5.0)
    e = jnp.exp(un)
    ei = 1.0 / e
    ch = 0.5 * (e + ei)
    sh = 0.5 * (e - ei)
    return jnp.where(_lane0(u.shape), ch, u * (sh / un))


def _l_logmap0(xx):
    x0 = jnp.maximum(_col0(xx), 1.0 + 1e-7)
    d = jnp.log(x0 + jnp.sqrt(x0 * x0 - 1.0))
    nr = jnp.maximum(_tailnorm(xx), 1e-7)
    return jnp.where(_lane0(xx.shape), 0.0, xx * (d / nr))


def _s_expmap0(u):
    import numpy as _np
    un = jnp.clip(_tailnorm(u), 1e-7, _np.pi - 1e-2)
    return jnp.where(_lane0(u.shape), jnp.cos(un), u * (jnp.sin(un) / un))


def _arccos(x):
    import numpy as _np
    ax = jnp.abs(x)
    small = ax <= 0.5
    z = jnp.where(small, x * x, 0.5 * (1.0 - ax))
    s = jnp.where(small, ax, jnp.sqrt(z))
    r = (z * (0.16666586697 + z * (-0.042743422091 + z * (-0.0086563630030)))
         / (1.0 + z * (-0.70662963390)))
    asn = s + s * r
    sgn_asn = jnp.where(x < 0.0, -asn, asn)
    small_th = jnp.float32(_np.pi / 2) - sgn_asn
    big_th = jnp.where(x < 0.0, jnp.float32(_np.pi) - 2.0 * asn, 2.0 * asn)
    return jnp.where(small, small_th, big_th)


def _s_logmap0(xx):
    x0 = jnp.clip(_col0(xx), -1.0 + 1e-6, 1.0 - 1e-6)
    th = _arccos(x0)
    nr = jnp.maximum(_tailnorm(xx), 1e-7)
    return jnp.where(_lane0(xx.shape), 0.0, xx * (th / nr))


def _l_proju(x, v):
    inner = jnp.sum(jnp.where(_lane0(x.shape), -x * v, x * v),
                    axis=-1, keepdims=True)
    return v + inner * x


def _l_transp0back(x, v):
    x0 = _col0(x)
    v0 = _col0(v)
    xo = x + jnp.where(_lane0(x.shape), 1.0, 0.0)
    return v + (v0 / (1.0 + x0)) * xo


def _s_proju(x, v):
    return v - jnp.sum(x * v, axis=-1, keepdims=True) * x


def _s_transp0back(x, v):
    x0 = _col0(x)
    v0 = _col0(v)
    xo = x + jnp.where(_lane0(x.shape), 1.0, 0.0)
    return v - (v0 / jnp.maximum(1.0 + x0, 1e-5)) * xo


def _mlp(t, W1, b1, W2, b2):
    return _dot(jnp.maximum(_dot(t, W1) + b1, 0.0), W2) + b2


def _node_spec():
    return pl.BlockSpec((_BLK, _D), lambda i: (i, 0))


def _pair_spec(width):
    return pl.BlockSpec((_NC, _BLK, width), lambda i: (0, i, 0))


def _full_spec(shape):
    nd = len(shape)
    return pl.BlockSpec(shape, lambda i: (0,) * nd)


def _tc1_body(tok, eW1, eb1, eW2, eb2, hW1, hb1, hW2, hb2,
              sW1, sb1, sW2, sb2, hE_o, hH_o, hS_o):
    t = tok[...]
    hE_o[...] = _mlp(t, eW1[...], eb1[...], eW2[...], eb2[...])
    hH_o[...] = _mlp(t, hW1[...], hb1[...], hW2[...], hb2[...])
    hS_o[...] = _mlp(t, sW1[...], sb1[...], sW2[...], sb2[...])


def _tc1(tok, eW1, eb1, eW2, eb2, hW1, hb1, hW2, hb2, sW1, sb1, sW2, sb2):
    ws = [eW1, eb1, eW2, eb2, hW1, hb1, hW2, hb2, sW1, sb1, sW2, sb2]
    return pl.pallas_call(
        _tc1_body,
        grid=(_GRID,),
        in_specs=[_node_spec()] + [_full_spec(w.shape) for w in ws],
        out_specs=[_node_spec()] * 3,
        out_shape=[jax.ShapeDtypeStruct((_N, _D), jnp.float32)] * 3,
    )(tok, *ws)


def _tc2_body(hE, hH, hS, aE, aH, aS, degn, Wh0,
              xE_o, xH_o, xS_o, u_o, dinv_o):
    dinv = 1.0 / (1.0 + degn[...])
    dinv_o[...] = jnp.broadcast_to(dinv, (_BLK, _D))
    xE_o[...] = (hE[...] + aE[...][0] + aE[...][1]) * dinv
    xH = _l_expmap0((hH[...] + aH[...][0] + aH[...][1]) * dinv)
    xH_o[...] = xH
    xS_o[...] = _s_expmap0((hS[...] + aS[...][0] + aS[...][1]) * dinv)
    u_o[...] = _dot(_l_logmap0(xH), Wh0[...])


def _tc2(hE, hH, hS, aE, aH, aS, degn, Wh0):
    return pl.pallas_call(
        _tc2_body,
        grid=(_GRID,),
        in_specs=[_node_spec()] * 3 + [_pair_spec(_D)] * 3
        + [pl.BlockSpec((_BLK, 1), lambda i: (i, 0)), _full_spec(Wh0.shape)],
        out_specs=[_node_spec()] * 5,
        out_shape=[jax.ShapeDtypeStruct((_N, _D), jnp.float32)] * 5,
    )(hE, hH, hS, aE, aH, aS, degn, Wh0)


def _tc3_body(xS, u, aU, dinv, WsA, WsB, xH_o, v_o):
    xH = _l_expmap0((u[...] + aU[...][0] + aU[...][1]) * dinv[...])
    xH_o[...] = xH
    v_o[...] = (_dot(_s_logmap0(xS[...]), WsA[...])
                + _dot(_l_logmap0(xH), WsB[...]))


def _tc3(xS, u, aU, dinv, WsA, WsB):
    return pl.pallas_call(
        _tc3_body,
        grid=(_GRID,),
        in_specs=[_node_spec(), _node_spec(), _pair_spec(_D), _node_spec(),
                  _full_spec(WsA.shape), _full_spec(WsB.shape)],
        out_specs=[_node_spec()] * 2,
        out_shape=[jax.ShapeDtypeStruct((_N, _D), jnp.float32)] * 2,
    )(xS, u, aU, dinv, WsA, WsB)


def _tc4_body(xE, xH, v, aV, dinv, P1a, P1b, P1c, pb1, P2, pb2, WhN,
              xE_o, xS_o, u_o):
    xS = _s_expmap0((v[...] + aV[...][0] + aV[...][1]) * dinv[...])
    xS_o[...] = xS
    xEi = xE[...]
    xHi = xH[...]
    HE = _l_transp0back(xHi, _l_proju(xHi, xEi))
    SE = _s_transp0back(xS, _s_proju(xS, xEi))
    hmid = jnp.maximum(_dot(xEi, P1a[...]) + _dot(HE, P1b[...])
                       + _dot(SE, P1c[...]) + pb1[...], 0.0)
    xE_o[...] = _dot(hmid, P2[...]) + pb2[...]
    u_o[...] = _dot(_l_logmap0(xHi), WhN[...])


def _tc4(xE, xH, v, aV, dinv, P1a, P1b, P1c, pb1, P2, pb2, WhN):
    ws = [P1a, P1b, P1c, pb1, P2, pb2, WhN]
    return pl.pallas_call(
        _tc4_body,
        grid=(_GRID,),
        in_specs=[_node_spec(), _node_spec(), _node_spec(), _pair_spec(_D),
                  _node_spec()] + [_full_spec(w.shape) for w in ws],
        out_specs=[_node_spec()] * 3,
        out_shape=[jax.ShapeDtypeStruct((_N, _D), jnp.float32)] * 3,
    )(xE, xH, v, aV, dinv, *ws)


def kernel(x, tokens, eW1, eb1, eW2, eb2, hW1, hb1, hW2, hb2,
           sW1, sb1, sW2, sb2, Wh, Ws, P1, pb1, P2, pb2, edge_index):
    pad = _EP - _E
    srcp = jnp.concatenate(
        [edge_index[0], (jnp.arange(pad, dtype=jnp.int32) % _N)])
    dstp = jnp.concatenate(
        [edge_index[1],
         _N + (jnp.arange(pad, dtype=jnp.int32) % (_NP - _N))])
    src = srcp.reshape(_NW, _NGRP, _GC, _C)
    dst = dstp.reshape(_NW, _NGRP, _GC, _C)
    zeros_d = jnp.zeros((_NP, _D), jnp.float32)

    def agg(h, s3, d3, z):
        return _sc_agg(_D)(h, s3, d3, z)[:, :_N]

    degp = _sc_deg()(dstp.reshape(_NW, _NCHUNK, _C))
    degn = _tc_degsum(degp).reshape(_DROW * _D, 1)[:_N]

    hE, hH, hS = _tc1(tokens, eW1, eb1.reshape(1, -1), eW2, eb2.reshape(1, -1),
                      hW1, hb1.reshape(1, -1), hW2, hb2.reshape(1, -1),
                      sW1, sb1.reshape(1, -1), sW2, sb2.reshape(1, -1))
    a3 = _sc_agg3()(hE, hH, hS, src, dst, zeros_d)
    aE = a3[0, :, :_N]
    aH = a3[1, :, :_N]
    aS = a3[2, :, :_N]
    xE, xH, xS, u, dinv = _tc2(hE, hH, hS, aE, aH, aS, degn, Wh[0])

    for i in range(Wh.shape[0]):
        aU = agg(u, src, dst, zeros_d)
        xH, v = _tc3(xS, u, aU, dinv, Ws[i][:_D], Ws[i][_D:])
        aV = agg(v, src, dst, zeros_d)
        WhN = Wh[(i + 1) % Wh.shape[0]]
        xE, xS, u = _tc4(xE, xH, v, aV, dinv,
                         P1[i][:_D], P1[i][_D:2 * _D], P1[i][2 * _D:],
                         pb1[i].reshape(1, -1), P2[i], pb2[i].reshape(1, -1),
                         WhN)
    return (xE, xH, xS)
